# baseline (speedup 1.0000x reference)
import functools

import jax
import jax.numpy as jnp
from jax import lax
from jax.experimental import pallas as pl
from jax.experimental.pallas import tpu as pltpu
from jax.experimental.pallas import tpu_sc as plsc

_NUM_CORES = 2
_NUM_SUBCORES = 16
_NUM_WORKERS = _NUM_CORES * _NUM_SUBCORES
_LANES = 16
_ROW = 128


def _gather_body(b_per_w, idx_hbm, table_hbm, out_hbm, idx_v, r_v, rows_v, sem):
    wid = lax.axis_index("s") * _NUM_CORES + lax.axis_index("c")
    base = wid * b_per_w
    pltpu.sync_copy(idx_hbm.at[pl.ds(base, b_per_w)], idx_v)
    for g in range(b_per_w // _LANES):
        v = idx_v[pl.ds(g * _LANES, _LANES)]
        r_v[pl.ds(g * _LANES, _LANES)] = lax.shift_right_logical(v, 1)
    pltpu.async_copy(table_hbm.at[r_v], rows_v, sem).wait()
    pltpu.sync_copy(rows_v, out_hbm.at[pl.ds(base, b_per_w)])


def kernel(idx, table):
    B = idx.shape[0]
    V, D = table.shape
    b_per_w = B // _NUM_WORKERS

    idx32 = idx.astype(jnp.int32)
    table_w = table.reshape(V // 2, _ROW)

    mesh = plsc.VectorSubcoreMesh(
        core_axis_name="c", subcore_axis_name="s",
        num_cores=_NUM_CORES, num_subcores=_NUM_SUBCORES)
    grid_kernel = pl.kernel(
        functools.partial(_gather_body, b_per_w),
        out_type=jax.ShapeDtypeStruct((B, _ROW), jnp.float32),
        mesh=mesh,
        scratch_types=[
            pltpu.VMEM((b_per_w,), jnp.int32),
            pltpu.VMEM((b_per_w,), jnp.int32),
            pltpu.VMEM((b_per_w, _ROW), jnp.float32),
            pltpu.SemaphoreType.DMA,
        ],
    )
    lines = grid_kernel(idx32, table_w)
    odd = (idx32 & 1).astype(jnp.bool_)[:, None]
    return jnp.where(odd, lines[:, D:], lines[:, :D])


# speedup vs baseline: 1.5715x; 1.5715x over previous
import functools

import jax
import jax.numpy as jnp
from jax import lax
from jax.experimental import pallas as pl
from jax.experimental.pallas import tpu as pltpu
from jax.experimental.pallas import tpu_sc as plsc

_NC = 2
_NS = 16
_NW = _NC * _NS
_L = 16


def _body(b_per_w, idx_hbm, table_hbm, out_hbm, idx_v, t8_v, r_v, rows_v,
          ring_v, sem0, sem1):
    n_groups = b_per_w // _L
    wid = lax.axis_index("s") * _NC + lax.axis_index("c")
    base = wid * b_per_w
    pltpu.sync_copy(idx_hbm.at[pl.ds(base, b_per_w)], idx_v)
    for g in range(n_groups):
        v = idx_v[pl.ds(g * _L, _L)]
        t8_v[pl.ds(g * _L, _L)] = lax.bitwise_and(v, ~7)
        r_v[pl.ds(g * _L, _L)] = lax.bitwise_and(v, 7)

    iota = lax.iota(jnp.int32, _L)

    def fire(g, half, sem):
        v8 = t8_v[pl.ds(g * _L, _L)]
        for k in range(_L):
            pltpu.async_copy(
                table_hbm.at[pl.ds(pl.multiple_of(v8[k], 8), 8)],
                ring_v.at[half, k], sem)

    def drain(sem):
        for k in range(_L):
            pltpu.make_async_copy(
                table_hbm.at[pl.ds(0, 8)],
                ring_v.at[0, k], sem).wait()

    def select(g, half):
        jvec = iota + g * _L
        rvec = r_v[pl.ds(g * _L, _L)]
        hsplat = jnp.full((_L,), half, jnp.int32)
        for col in range(64):
            csplat = jnp.full((_L,), col, jnp.int32)
            vals = plsc.load_gather(ring_v, [hsplat, iota, rvec, csplat])
            plsc.store_scatter(rows_v, [jvec, csplat], vals)

    fire(0, 0, sem0)
    fire(1, 1, sem1)

    def loop_body(m, carry):
        g0 = m * 2
        drain(sem0)
        select(g0, 0)

        @pl.when(m < (n_groups // 2 - 1))
        def _():
            fire(g0 + 2, 0, sem0)

        drain(sem1)
        select(g0 + 1, 1)

        @pl.when(m < (n_groups // 2 - 1))
        def _():
            fire(g0 + 3, 1, sem1)

        return carry

    lax.fori_loop(0, n_groups // 2, loop_body, 0)
    pltpu.sync_copy(rows_v, out_hbm.at[pl.ds(base, b_per_w)])


def kernel(idx, table):
    B = idx.shape[0]
    V, D = table.shape
    b_per_w = B // _NW

    mesh = plsc.VectorSubcoreMesh(
        core_axis_name="c", subcore_axis_name="s",
        num_cores=_NC, num_subcores=_NS)
    grid_kernel = pl.kernel(
        functools.partial(_body, b_per_w),
        out_type=jax.ShapeDtypeStruct((B, D), jnp.float32),
        mesh=mesh,
        scratch_types=[
            pltpu.VMEM((b_per_w,), jnp.int32),
            pltpu.VMEM((b_per_w,), jnp.int32),
            pltpu.VMEM((b_per_w,), jnp.int32),
            pltpu.VMEM((b_per_w, D), jnp.float32),
            pltpu.VMEM((2, _L, 8, D), jnp.float32),
            pltpu.SemaphoreType.DMA,
            pltpu.SemaphoreType.DMA,
        ],
        compiler_params=pltpu.CompilerParams(needs_layout_passes=False),
    )
    return grid_kernel(idx.astype(jnp.int32), table)


# speedup vs baseline: 1.6131x; 1.0264x over previous
import functools

import jax
import jax.numpy as jnp
from jax import lax
from jax.experimental import pallas as pl
from jax.experimental.pallas import tpu as pltpu
from jax.experimental.pallas import tpu_sc as plsc

_NC = 2
_NS = 16
_NW = _NC * _NS
_L = 16


def _body(b_per_w, idx_hbm, table_hbm, out_hbm, idx_v, t8_v, r_v, rows_v,
          ring_v, sem0, sem1):
    n_groups = b_per_w // _L
    wid = lax.axis_index("s") * _NC + lax.axis_index("c")
    base = wid * b_per_w
    pltpu.sync_copy(idx_hbm.at[pl.ds(base, b_per_w)], idx_v)
    for g in range(n_groups):
        v = idx_v[pl.ds(g * _L, _L)]
        t8_v[pl.ds(g * _L, _L)] = lax.bitwise_and(v, ~7)
        r_v[pl.ds(g * _L, _L)] = lax.bitwise_and(v, 7)

    iota = lax.iota(jnp.int32, _L)

    def fire(g, half, sem):
        v8 = t8_v[pl.ds(g * _L, _L)]
        for k in range(_L):
            pltpu.async_copy(
                table_hbm.at[pl.ds(pl.multiple_of(v8[k], 8), 8)],
                ring_v.at[half, k], sem)

    def drain(sem):
        for k in range(_L):
            pltpu.make_async_copy(
                table_hbm.at[pl.ds(0, 8)],
                ring_v.at[0, k], sem).wait()

    def select(g, half):
        rvec = r_v[pl.ds(g * _L, _L)]
        for k in range(_L):
            r = rvec[k]
            j = g * _L + k
            for q in range(64 // _L):
                rows_v[j, pl.ds(q * _L, _L)] = (
                    ring_v[half, k, r, pl.ds(q * _L, _L)])

    fire(0, 0, sem0)
    fire(1, 1, sem1)

    def loop_body(m, carry):
        g0 = m * 2
        drain(sem0)
        select(g0, 0)

        @pl.when(m < (n_groups // 2 - 1))
        def _():
            fire(g0 + 2, 0, sem0)

        drain(sem1)
        select(g0 + 1, 1)

        @pl.when(m < (n_groups // 2 - 1))
        def _():
            fire(g0 + 3, 1, sem1)

        return carry

    lax.fori_loop(0, n_groups // 2, loop_body, 0)
    pltpu.sync_copy(rows_v, out_hbm.at[pl.ds(base, b_per_w)])


def kernel(idx, table):
    B = idx.shape[0]
    V, D = table.shape
    b_per_w = B // _NW

    mesh = plsc.VectorSubcoreMesh(
        core_axis_name="c", subcore_axis_name="s",
        num_cores=_NC, num_subcores=_NS)
    grid_kernel = pl.kernel(
        functools.partial(_body, b_per_w),
        out_type=jax.ShapeDtypeStruct((B, D), jnp.float32),
        mesh=mesh,
        scratch_types=[
            pltpu.VMEM((b_per_w,), jnp.int32),
            pltpu.VMEM((b_per_w,), jnp.int32),
            pltpu.VMEM((b_per_w,), jnp.int32),
            pltpu.VMEM((b_per_w, D), jnp.float32),
            pltpu.VMEM((2, _L, 8, D), jnp.float32),
            pltpu.SemaphoreType.DMA,
            pltpu.SemaphoreType.DMA,
        ],
    )
    return grid_kernel(idx.astype(jnp.int32), table)


# speedup vs baseline: 1.6181x; 1.0031x over previous
import functools

import jax
import jax.numpy as jnp
from jax import lax
from jax.experimental import layout as jax_layout
from jax.experimental import pallas as pl
from jax.experimental.pallas import tpu as pltpu
from jax.experimental.pallas import tpu_sc as plsc

_NC = 2
_NS = 16
_NW = _NC * _NS
_L = 16


def _body(b_per_w, idx_hbm, table_hbm, out_hbm, idx_v, t8_v, r_v, rows_v,
          ring_v, sem0, sem1):
    n_groups = b_per_w // _L
    wid = lax.axis_index("s") * _NC + lax.axis_index("c")
    base = wid * b_per_w
    pltpu.sync_copy(idx_hbm.at[pl.ds(base, b_per_w)], idx_v)
    for g in range(n_groups):
        v = idx_v[pl.ds(g * _L, _L)]
        t8_v[pl.ds(g * _L, _L)] = lax.bitwise_and(v, ~7)
        r_v[pl.ds(g * _L, _L)] = lax.bitwise_and(v, 7)

    iota = lax.iota(jnp.int32, _L)

    def fire(g, half, sem):
        v8 = t8_v[pl.ds(g * _L, _L)]
        for k in range(_L):
            pltpu.async_copy(
                table_hbm.at[pl.ds(pl.multiple_of(v8[k], 8), 8)],
                ring_v.at[half, k], sem)

    def drain(sem):
        for k in range(_L):
            pltpu.make_async_copy(
                table_hbm.at[pl.ds(0, 8)],
                ring_v.at[0, k], sem).wait()

    def select(g, half):
        rvec = r_v[pl.ds(g * _L, _L)]
        for k in range(_L):
            r = rvec[k]
            j = g * _L + k
            for q in range(64 // _L):
                rows_v[j, pl.ds(q * _L, _L)] = (
                    ring_v[half, k, r, pl.ds(q * _L, _L)])

    fire(0, 0, sem0)
    fire(1, 1, sem1)

    def loop_body(m, carry):
        g0 = m * 2
        drain(sem0)
        select(g0, 0)

        @pl.when(m < (n_groups // 2 - 1))
        def _():
            fire(g0 + 2, 0, sem0)

        drain(sem1)
        select(g0 + 1, 1)

        @pl.when(m < (n_groups // 2 - 1))
        def _():
            fire(g0 + 3, 1, sem1)

        return carry

    lax.fori_loop(0, n_groups // 2, loop_body, 0)
    pltpu.sync_copy(rows_v, out_hbm.at[pl.ds(base, b_per_w)])


def kernel(idx, table):
    B = idx.shape[0]
    V, D = table.shape
    b_per_w = B // _NW

    table_c = jax_layout.with_layout_constraint(
        table, jax_layout.Layout((1, 0)))
    table_c = lax.optimization_barrier(table_c)

    mesh = plsc.VectorSubcoreMesh(
        core_axis_name="c", subcore_axis_name="s",
        num_cores=_NC, num_subcores=_NS)
    grid_kernel = pl.kernel(
        functools.partial(_body, b_per_w),
        out_type=jax.ShapeDtypeStruct((B, D), jnp.float32),
        mesh=mesh,
        scratch_types=[
            pltpu.VMEM((b_per_w,), jnp.int32),
            pltpu.VMEM((b_per_w,), jnp.int32),
            pltpu.VMEM((b_per_w,), jnp.int32),
            pltpu.VMEM((b_per_w, D), jnp.float32),
            pltpu.VMEM((2, _L, 8, D), jnp.float32),
            pltpu.SemaphoreType.DMA,
            pltpu.SemaphoreType.DMA,
        ],
    )
    return grid_kernel(idx.astype(jnp.int32), table_c)
